# baseline (speedup 1.0000x reference)
import functools

import jax
import jax.numpy as jnp
from jax import lax
from jax.experimental import pallas as pl
from jax.experimental.pallas import tpu as pltpu
from jax.experimental.pallas import tpu_sc as plsc

N = 10000
NP = 10240
E = 320000
F = 128
D1 = 128
D2 = 64
D3 = 32
G = 64

NC = 2
NS = 16
NW = NC * NS
EW = E // NW
K = 80
ROWS_W = NP // NS
RB = 80

_mesh = plsc.VectorSubcoreMesh(
    core_axis_name="c", subcore_axis_name="s", num_cores=NC, num_subcores=NS)


def _make_sc_layer(D, KL, nbuf):
  nchunk = EW // KL
  ngroup = nchunk // nbuf
  tail = nchunk % nbuf

  @functools.partial(
      pl.kernel,
      out_type=jax.ShapeDtypeStruct((NC, NP, D), jnp.float32),
      mesh=_mesh,
      scratch_types=[
          pltpu.VMEM((EW,), jnp.int32),
          pltpu.VMEM((EW,), jnp.int32),
          pltpu.VMEM((nbuf, KL, D), jnp.float32),
          pltpu.VMEM_SHARED((NP, D), jnp.float32),
      ] + [pltpu.SemaphoreType.DMA] * nbuf,
      compiler_params=pltpu.CompilerParams(use_tc_tiling_on_sc=False),
  )
  def sc_layer(src_hbm, dst_hbm, g_hbm, zero_hbm, out_hbm,
               src_v, dst_v, rows_v, acc_sh, *sems):
    c = lax.axis_index("c")
    s = lax.axis_index("s")
    wid = c * NS + s
    stripe = s * ROWS_W
    ebase = wid * EW

    def start_gather(b, base):
      pltpu.async_copy(
          g_hbm.at[src_v.at[pl.ds(base, KL)]], rows_v.at[b], sems[b])

    def drain_scatter(b, base):
      pltpu.make_async_copy(
          g_hbm.at[pl.ds(0, KL)], rows_v.at[b], sems[b]).wait()
      pltpu.sync_copy(
          rows_v.at[b], acc_sh.at[dst_v.at[pl.ds(base, KL)]], add=True)

    pltpu.sync_copy(zero_hbm, rows_v.at[0])
    for r in range(ROWS_W // KL):
      pltpu.sync_copy(rows_v.at[0], acc_sh.at[pl.ds(stripe + r * KL, KL)])
    pltpu.sync_copy(src_hbm.at[pl.ds(ebase, EW)], src_v)
    pltpu.sync_copy(dst_hbm.at[pl.ds(ebase, EW)], dst_v)
    plsc.subcore_barrier()

    for b in range(nbuf):
      start_gather(b, b * KL)

    def outer(j, carry):
      for b in range(nbuf):
        base = (j * nbuf + b) * KL
        drain_scatter(b, base)
        start_gather(b, base + nbuf * KL)
      return carry

    lax.fori_loop(0, ngroup - 1, outer, 0)

    for b in range(nbuf):
      base = ((ngroup - 1) * nbuf + b) * KL
      drain_scatter(b, base)
      if b < tail:
        start_gather(b, base + nbuf * KL)
    for b in range(tail):
      drain_scatter(b, (ngroup * nbuf + b) * KL)
    plsc.subcore_barrier()

    for r in range(ROWS_W // KL):
      pltpu.sync_copy(acc_sh.at[pl.ds(stripe + r * KL, KL)], rows_v.at[0])
      pltpu.sync_copy(rows_v.at[0], out_hbm.at[c, pl.ds(stripe + r * KL, KL)])

  return sc_layer


@functools.partial(
    pl.kernel,
    out_type=jax.ShapeDtypeStruct((NC, NP, 16), jnp.float32),
    mesh=_mesh,
    scratch_types=[
        pltpu.VMEM((EW,), jnp.int32),
        pltpu.VMEM((K, 16), jnp.float32),
        pltpu.VMEM((RB, 16), jnp.float32),
        pltpu.VMEM_SHARED((NP, 16), jnp.float32),
    ],
    compiler_params=pltpu.CompilerParams(use_tc_tiling_on_sc=False),
)
def _sc_degree(dst_hbm, ones_hbm, zero_hbm, out_hbm,
               dst_v, ones_v, stg_v, acc_sh):
  c = lax.axis_index("c")
  s = lax.axis_index("s")
  wid = c * NS + s
  stripe = s * ROWS_W

  pltpu.sync_copy(zero_hbm, stg_v)
  for r in range(ROWS_W // RB):
    pltpu.sync_copy(stg_v, acc_sh.at[pl.ds(stripe + r * RB, RB)])
  pltpu.sync_copy(ones_hbm, ones_v)
  pltpu.sync_copy(dst_hbm.at[pl.ds(wid * EW, EW)], dst_v)
  plsc.subcore_barrier()

  def body(i, carry):
    pltpu.sync_copy(ones_v, acc_sh.at[dst_v.at[pl.ds(i * K, K)]], add=True)
    return carry

  lax.fori_loop(0, EW // K, body, 0)
  plsc.subcore_barrier()

  for r in range(ROWS_W // RB):
    pltpu.sync_copy(acc_sh.at[pl.ds(stripe + r * RB, RB)], stg_v)
    pltpu.sync_copy(stg_v, out_hbm.at[c, pl.ds(stripe + r * RB, RB)])


_RB_TC = 256
_GRID = NP // _RB_TC


def _tc_first(dega, x_pad, W1):
  def body(deg_ref, x_ref, w_ref, g_ref, dis_ref):
    deg = 1.0 + deg_ref[0, :, 0:1] + deg_ref[1, :, 0:1]
    dis = lax.rsqrt(deg)
    dis_ref[...] = dis
    t = jnp.dot(x_ref[...], w_ref[...], preferred_element_type=jnp.float32)
    g_ref[...] = t * dis

  return pl.pallas_call(
      body,
      grid=(_GRID,),
      in_specs=[
          pl.BlockSpec((NC, _RB_TC, 16), lambda i: (0, i, 0)),
          pl.BlockSpec((_RB_TC, F), lambda i: (i, 0)),
          pl.BlockSpec((F, D1), lambda i: (0, 0)),
      ],
      out_specs=[
          pl.BlockSpec((_RB_TC, D1), lambda i: (i, 0)),
          pl.BlockSpec((_RB_TC, 1), lambda i: (i, 0)),
      ],
      out_shape=[
          jax.ShapeDtypeStruct((NP, D1), jnp.float32),
          jax.ShapeDtypeStruct((NP, 1), jnp.float32),
      ],
  )(dega, x_pad, W1)


def _tc_layer(S, g, dis, b, Wn, D, Dn):
  def body(s_ref, g_ref, dis_ref, b_ref, w_ref, go_ref):
    dis = dis_ref[...]
    h = jnp.maximum(dis * (s_ref[0] + s_ref[1] + g_ref[...]) + b_ref[...], 0.0)
    go_ref[...] = jnp.dot(
        h, w_ref[...], preferred_element_type=jnp.float32) * dis

  return pl.pallas_call(
      body,
      grid=(_GRID,),
      in_specs=[
          pl.BlockSpec((NC, _RB_TC, D), lambda i: (0, i, 0)),
          pl.BlockSpec((_RB_TC, D), lambda i: (i, 0)),
          pl.BlockSpec((_RB_TC, 1), lambda i: (i, 0)),
          pl.BlockSpec((1, D), lambda i: (0, 0)),
          pl.BlockSpec((D, Dn), lambda i: (0, 0)),
      ],
      out_specs=pl.BlockSpec((_RB_TC, Dn), lambda i: (i, 0)),
      out_shape=jax.ShapeDtypeStruct((NP, Dn), jnp.float32),
  )(S, g, dis, b, Wn)


def _tc_pool_classify(S, g, dis, b, batch3, Wc, bc):
  def body(s_ref, g_ref, dis_ref, b_ref, bt_ref, wc_ref, bc_ref, o_ref,
           pooled_ref, cnt_ref):
    i = pl.program_id(0)
    dis = dis_ref[...]
    h = jnp.maximum(dis * (s_ref[0] + s_ref[1] + g_ref[...]) + b_ref[...], 0.0)
    bt = bt_ref[0]
    gids = lax.broadcasted_iota(jnp.int32, (G, _RB_TC), 0)
    oh = (gids == bt).astype(jnp.float32)
    pooled_p = jnp.dot(oh, h, preferred_element_type=jnp.float32)
    cnt_p = jnp.sum(oh, axis=1, keepdims=True)

    @pl.when(i == 0)
    def _():
      pooled_ref[...] = jnp.zeros_like(pooled_ref)
      cnt_ref[...] = jnp.zeros_like(cnt_ref)

    pooled_ref[...] += pooled_p
    cnt_ref[...] += cnt_p

    @pl.when(i == _GRID - 1)
    def _():
      p = pooled_ref[...] / jnp.maximum(cnt_ref[...], 1.0)
      logits = jnp.dot(p, wc_ref[...], preferred_element_type=jnp.float32)
      logits = logits + bc_ref[...]
      m = jnp.max(logits, axis=1, keepdims=True)
      z = logits - m
      o_ref[...] = z - jnp.log(jnp.sum(jnp.exp(z), axis=1, keepdims=True))

  return pl.pallas_call(
      body,
      grid=(_GRID,),
      in_specs=[
          pl.BlockSpec((NC, _RB_TC, D3), lambda i: (0, i, 0)),
          pl.BlockSpec((_RB_TC, D3), lambda i: (i, 0)),
          pl.BlockSpec((_RB_TC, 1), lambda i: (i, 0)),
          pl.BlockSpec((1, D3), lambda i: (0, 0)),
          pl.BlockSpec((1, 1, _RB_TC), lambda i: (i, 0, 0)),
          pl.BlockSpec((D3, 2), lambda i: (0, 0)),
          pl.BlockSpec((1, 2), lambda i: (0, 0)),
      ],
      out_specs=pl.BlockSpec((G, 2), lambda i: (0, 0)),
      out_shape=jax.ShapeDtypeStruct((G, 2), jnp.float32),
      scratch_shapes=[
          pltpu.VMEM((G, D3), jnp.float32),
          pltpu.VMEM((G, 1), jnp.float32),
      ],
  )(S, g, dis, b, batch3, Wc, bc)


_sc_l1 = _make_sc_layer(D1, K, 2)
_sc_l2 = _make_sc_layer(D2, K, 5)
_sc_l3 = _make_sc_layer(D3, K, 5)


@jax.jit
def kernel(x, edge_index, batch, W1, b1, W2, b2, W3, b3, Wc, bc):
  src = edge_index[0]
  dst = edge_index[1]
  x_pad = jnp.zeros((NP, F), jnp.float32).at[:N].set(x)
  batch_pad = jnp.concatenate(
      [batch, jnp.full((NP - N,), G, jnp.int32)]).reshape(_GRID, 1, _RB_TC)

  ones80 = jnp.ones((K, 16), jnp.float32)
  zero16 = jnp.zeros((RB, 16), jnp.float32)
  z1 = jnp.zeros((K, D1), jnp.float32)
  z2 = jnp.zeros((K, D2), jnp.float32)
  z3 = jnp.zeros((K, D3), jnp.float32)

  dega = _sc_degree(dst, ones80, zero16)
  g1, dis = _tc_first(dega, x_pad, W1)
  S1 = _sc_l1(src, dst, g1, z1)
  g2 = _tc_layer(S1, g1, dis, b1.reshape(1, D1), W2, D1, D2)
  S2 = _sc_l2(src, dst, g2, z2)
  g3 = _tc_layer(S2, g2, dis, b2.reshape(1, D2), W3, D2, D3)
  S3 = _sc_l3(src, dst, g3, z3)
  return _tc_pool_classify(S3, g3, dis, b3.reshape(1, D3), batch_pad,
                           Wc, bc.reshape(1, 2))


# speedup vs baseline: 1.0607x; 1.0607x over previous
import functools

import jax
import jax.numpy as jnp
from jax import lax
from jax.experimental import pallas as pl
from jax.experimental.pallas import tpu as pltpu
from jax.experimental.pallas import tpu_sc as plsc

N = 10000
NP = 10240
E = 320000
F = 128
D1 = 128
D2 = 64
D3 = 32
G = 64

NC = 2
NS = 16
NW = NC * NS
EW = E // NW
K = 80
ROWS_W = NP // NS
RB = 80

_mesh = plsc.VectorSubcoreMesh(
    core_axis_name="c", subcore_axis_name="s", num_cores=NC, num_subcores=NS)


def _make_sc_layer(D, KL, nbuf):
  nchunk = EW // KL
  ngroup = nchunk // nbuf
  tail = nchunk % nbuf

  @functools.partial(
      pl.kernel,
      out_type=jax.ShapeDtypeStruct((NC, NP, D), jnp.float32),
      mesh=_mesh,
      scratch_types=[
          pltpu.VMEM((EW,), jnp.int32),
          pltpu.VMEM((EW,), jnp.int32),
          pltpu.VMEM((nbuf, KL, D), jnp.float32),
          pltpu.VMEM_SHARED((NP, D), jnp.float32),
      ] + [pltpu.SemaphoreType.DMA] * nbuf,
      compiler_params=pltpu.CompilerParams(use_tc_tiling_on_sc=False),
  )
  def sc_layer(src_hbm, dst_hbm, g_hbm, zero_hbm, out_hbm,
               src_v, dst_v, rows_v, acc_sh, *sems):
    c = lax.axis_index("c")
    s = lax.axis_index("s")
    wid = c * NS + s
    stripe = s * ROWS_W
    ebase = wid * EW

    def start_gather(b, base):
      pltpu.async_copy(
          g_hbm.at[src_v.at[pl.ds(base, KL)]], rows_v.at[b], sems[b])

    def drain_scatter(b, base):
      pltpu.make_async_copy(
          g_hbm.at[pl.ds(0, KL)], rows_v.at[b], sems[b]).wait()
      pltpu.sync_copy(
          rows_v.at[b], acc_sh.at[dst_v.at[pl.ds(base, KL)]], add=True)

    pltpu.sync_copy(zero_hbm, rows_v.at[0])
    for r in range(ROWS_W // KL):
      pltpu.sync_copy(rows_v.at[0], acc_sh.at[pl.ds(stripe + r * KL, KL)])
    pltpu.sync_copy(src_hbm.at[pl.ds(ebase, EW)], src_v)
    pltpu.sync_copy(dst_hbm.at[pl.ds(ebase, EW)], dst_v)
    plsc.subcore_barrier()

    for b in range(nbuf):
      start_gather(b, b * KL)

    def outer(j, carry):
      for b in range(nbuf):
        base = (j * nbuf + b) * KL
        drain_scatter(b, base)
        start_gather(b, base + nbuf * KL)
      return carry

    lax.fori_loop(0, ngroup - 1, outer, 0)

    for b in range(nbuf):
      base = ((ngroup - 1) * nbuf + b) * KL
      drain_scatter(b, base)
      if b < tail:
        start_gather(b, base + nbuf * KL)
    for b in range(tail):
      drain_scatter(b, (ngroup * nbuf + b) * KL)
    plsc.subcore_barrier()

    for r in range(ROWS_W // KL):
      pltpu.sync_copy(acc_sh.at[pl.ds(stripe + r * KL, KL)], rows_v.at[0])
      pltpu.sync_copy(rows_v.at[0], out_hbm.at[c, pl.ds(stripe + r * KL, KL)])

  return sc_layer


@functools.partial(
    pl.kernel,
    out_type=jax.ShapeDtypeStruct((NC, NP, 16), jnp.float32),
    mesh=_mesh,
    scratch_types=[
        pltpu.VMEM((EW,), jnp.int32),
        pltpu.VMEM((K, 16), jnp.float32),
        pltpu.VMEM((RB, 16), jnp.float32),
        pltpu.VMEM_SHARED((NP, 16), jnp.float32),
    ],
    compiler_params=pltpu.CompilerParams(use_tc_tiling_on_sc=False),
)
def _sc_degree(dst_hbm, ones_hbm, zero_hbm, out_hbm,
               dst_v, ones_v, stg_v, acc_sh):
  c = lax.axis_index("c")
  s = lax.axis_index("s")
  wid = c * NS + s
  stripe = s * ROWS_W

  pltpu.sync_copy(zero_hbm, stg_v)
  for r in range(ROWS_W // RB):
    pltpu.sync_copy(stg_v, acc_sh.at[pl.ds(stripe + r * RB, RB)])
  pltpu.sync_copy(ones_hbm, ones_v)
  pltpu.sync_copy(dst_hbm.at[pl.ds(wid * EW, EW)], dst_v)
  plsc.subcore_barrier()

  def body(i, carry):
    pltpu.sync_copy(ones_v, acc_sh.at[dst_v.at[pl.ds(i * K, K)]], add=True)
    return carry

  lax.fori_loop(0, EW // K, body, 0)
  plsc.subcore_barrier()

  for r in range(ROWS_W // RB):
    pltpu.sync_copy(acc_sh.at[pl.ds(stripe + r * RB, RB)], stg_v)
    pltpu.sync_copy(stg_v, out_hbm.at[c, pl.ds(stripe + r * RB, RB)])


_RB_TC = 256
_GRID = NP // _RB_TC


def _tc_first(dega, x_pad, W1):
  def body(deg_ref, x_ref, w_ref, g_ref, dis_ref):
    deg = 1.0 + deg_ref[0, :, 0:1] + deg_ref[1, :, 0:1]
    dis = lax.rsqrt(deg)
    dis_ref[...] = dis
    t = jnp.dot(x_ref[...], w_ref[...], preferred_element_type=jnp.float32)
    g_ref[...] = t * dis

  return pl.pallas_call(
      body,
      grid=(_GRID,),
      in_specs=[
          pl.BlockSpec((NC, _RB_TC, 16), lambda i: (0, i, 0)),
          pl.BlockSpec((_RB_TC, F), lambda i: (i, 0)),
          pl.BlockSpec((F, D1), lambda i: (0, 0)),
      ],
      out_specs=[
          pl.BlockSpec((_RB_TC, D1), lambda i: (i, 0)),
          pl.BlockSpec((_RB_TC, 1), lambda i: (i, 0)),
      ],
      out_shape=[
          jax.ShapeDtypeStruct((NP, D1), jnp.float32),
          jax.ShapeDtypeStruct((NP, 1), jnp.float32),
      ],
  )(dega, x_pad, W1)


def _tc_layer(S, g, dis, b, Wn, D, Dn):
  def body(s_ref, g_ref, dis_ref, b_ref, w_ref, go_ref):
    dis = dis_ref[...]
    h = jnp.maximum(dis * (s_ref[0] + s_ref[1] + g_ref[...]) + b_ref[...], 0.0)
    go_ref[...] = jnp.dot(
        h, w_ref[...], preferred_element_type=jnp.float32) * dis

  return pl.pallas_call(
      body,
      grid=(_GRID,),
      in_specs=[
          pl.BlockSpec((NC, _RB_TC, D), lambda i: (0, i, 0)),
          pl.BlockSpec((_RB_TC, D), lambda i: (i, 0)),
          pl.BlockSpec((_RB_TC, 1), lambda i: (i, 0)),
          pl.BlockSpec((1, D), lambda i: (0, 0)),
          pl.BlockSpec((D, Dn), lambda i: (0, 0)),
      ],
      out_specs=pl.BlockSpec((_RB_TC, Dn), lambda i: (i, 0)),
      out_shape=jax.ShapeDtypeStruct((NP, Dn), jnp.float32),
  )(S, g, dis, b, Wn)


def _tc_pool_classify(S, g, dis, b, batch3, Wc, bc):
  def body(s_ref, g_ref, dis_ref, b_ref, bt_ref, wc_ref, bc_ref, o_ref,
           pooled_ref, cnt_ref):
    i = pl.program_id(0)
    dis = dis_ref[...]
    h = jnp.maximum(dis * (s_ref[0] + s_ref[1] + g_ref[...]) + b_ref[...], 0.0)
    bt = bt_ref[0]
    gids = lax.broadcasted_iota(jnp.int32, (G, _RB_TC), 0)
    oh = (gids == bt).astype(jnp.float32)
    pooled_p = jnp.dot(oh, h, preferred_element_type=jnp.float32)
    cnt_p = jnp.sum(oh, axis=1, keepdims=True)

    @pl.when(i == 0)
    def _():
      pooled_ref[...] = jnp.zeros_like(pooled_ref)
      cnt_ref[...] = jnp.zeros_like(cnt_ref)

    pooled_ref[...] += pooled_p
    cnt_ref[...] += cnt_p

    @pl.when(i == _GRID - 1)
    def _():
      p = pooled_ref[...] / jnp.maximum(cnt_ref[...], 1.0)
      logits = jnp.dot(p, wc_ref[...], preferred_element_type=jnp.float32)
      logits = logits + bc_ref[...]
      m = jnp.max(logits, axis=1, keepdims=True)
      z = logits - m
      o_ref[...] = z - jnp.log(jnp.sum(jnp.exp(z), axis=1, keepdims=True))

  return pl.pallas_call(
      body,
      grid=(_GRID,),
      in_specs=[
          pl.BlockSpec((NC, _RB_TC, D3), lambda i: (0, i, 0)),
          pl.BlockSpec((_RB_TC, D3), lambda i: (i, 0)),
          pl.BlockSpec((_RB_TC, 1), lambda i: (i, 0)),
          pl.BlockSpec((1, D3), lambda i: (0, 0)),
          pl.BlockSpec((1, 1, _RB_TC), lambda i: (i, 0, 0)),
          pl.BlockSpec((D3, 2), lambda i: (0, 0)),
          pl.BlockSpec((1, 2), lambda i: (0, 0)),
      ],
      out_specs=pl.BlockSpec((G, 2), lambda i: (0, 0)),
      out_shape=jax.ShapeDtypeStruct((G, 2), jnp.float32),
      scratch_shapes=[
          pltpu.VMEM((G, D3), jnp.float32),
          pltpu.VMEM((G, 1), jnp.float32),
      ],
  )(S, g, dis, b, batch3, Wc, bc)


K1 = 40
_sc_l1 = _make_sc_layer(D1, K1, 5)
_sc_l2 = _make_sc_layer(D2, K, 5)
_sc_l3 = _make_sc_layer(D3, K, 5)


@jax.jit
def kernel(x, edge_index, batch, W1, b1, W2, b2, W3, b3, Wc, bc):
  src = edge_index[0]
  dst = edge_index[1]
  x_pad = jnp.zeros((NP, F), jnp.float32).at[:N].set(x)
  batch_pad = jnp.concatenate(
      [batch, jnp.full((NP - N,), G, jnp.int32)]).reshape(_GRID, 1, _RB_TC)

  ones80 = jnp.ones((K, 16), jnp.float32)
  zero16 = jnp.zeros((RB, 16), jnp.float32)
  z1 = jnp.zeros((K1, D1), jnp.float32)
  z2 = jnp.zeros((K, D2), jnp.float32)
  z3 = jnp.zeros((K, D3), jnp.float32)

  dega = _sc_degree(dst, ones80, zero16)
  g1, dis = _tc_first(dega, x_pad, W1)
  S1 = _sc_l1(src, dst, g1, z1)
  g2 = _tc_layer(S1, g1, dis, b1.reshape(1, D1), W2, D1, D2)
  S2 = _sc_l2(src, dst, g2, z2)
  g3 = _tc_layer(S2, g2, dis, b2.reshape(1, D2), W3, D2, D3)
  S3 = _sc_l3(src, dst, g3, z3)
  return _tc_pool_classify(S3, g3, dis, b3.reshape(1, D3), batch_pad,
                           Wc, bc.reshape(1, 2))


# speedup vs baseline: 1.0875x; 1.0252x over previous
import functools

import jax
import jax.numpy as jnp
from jax import lax
from jax.experimental import pallas as pl
from jax.experimental.pallas import tpu as pltpu
from jax.experimental.pallas import tpu_sc as plsc

N = 10000
NP = 10240
E = 320000
F = 128
D1 = 128
D2 = 64
D3 = 32
G = 64

NC = 2
NS = 16
NW = NC * NS
EW = E // NW
K = 80
ROWS_W = NP // NS
RB = 80

_mesh = plsc.VectorSubcoreMesh(
    core_axis_name="c", subcore_axis_name="s", num_cores=NC, num_subcores=NS)


def _make_sc_layer(D, KL, nbuf):
  nchunk = EW // KL
  ngroup = nchunk // nbuf
  tail = nchunk % nbuf

  @functools.partial(
      pl.kernel,
      out_type=jax.ShapeDtypeStruct((NC, NP, D), jnp.float32),
      mesh=_mesh,
      scratch_types=[
          pltpu.VMEM((EW,), jnp.int32),
          pltpu.VMEM((EW,), jnp.int32),
          pltpu.VMEM((nbuf, KL, D), jnp.float32),
          pltpu.VMEM_SHARED((NP, D), jnp.float32),
      ] + [pltpu.SemaphoreType.DMA] * nbuf,
      compiler_params=pltpu.CompilerParams(use_tc_tiling_on_sc=False),
  )
  def sc_layer(src_hbm, dst_hbm, g_hbm, zero_hbm, out_hbm,
               src_v, dst_v, rows_v, acc_sh, *sems):
    c = lax.axis_index("c")
    s = lax.axis_index("s")
    wid = c * NS + s
    stripe = s * ROWS_W
    ebase = wid * EW

    def start_gather(b, base):
      pltpu.async_copy(
          g_hbm.at[src_v.at[pl.ds(base, KL)]], rows_v.at[b], sems[b])

    def drain_scatter(b, base):
      pltpu.make_async_copy(
          g_hbm.at[pl.ds(0, KL)], rows_v.at[b], sems[b]).wait()
      pltpu.sync_copy(
          rows_v.at[b], acc_sh.at[dst_v.at[pl.ds(base, KL)]], add=True)

    pltpu.sync_copy(zero_hbm, rows_v.at[0])
    for r in range(ROWS_W // KL):
      pltpu.sync_copy(rows_v.at[0], acc_sh.at[pl.ds(stripe + r * KL, KL)])
    pltpu.sync_copy(src_hbm.at[pl.ds(ebase, EW)], src_v)
    pltpu.sync_copy(dst_hbm.at[pl.ds(ebase, EW)], dst_v)
    plsc.subcore_barrier()

    for b in range(nbuf):
      start_gather(b, b * KL)

    def outer(j, carry):
      for b in range(nbuf):
        base = (j * nbuf + b) * KL
        drain_scatter(b, base)
        start_gather(b, base + nbuf * KL)
      return carry

    lax.fori_loop(0, ngroup - 1, outer, 0)

    for b in range(nbuf):
      base = ((ngroup - 1) * nbuf + b) * KL
      drain_scatter(b, base)
      if b < tail:
        start_gather(b, base + nbuf * KL)
    for b in range(tail):
      drain_scatter(b, (ngroup * nbuf + b) * KL)
    plsc.subcore_barrier()

    for r in range(ROWS_W // KL):
      pltpu.sync_copy(acc_sh.at[pl.ds(stripe + r * KL, KL)], rows_v.at[0])
      pltpu.sync_copy(rows_v.at[0], out_hbm.at[c, pl.ds(stripe + r * KL, KL)])

  return sc_layer


@functools.partial(
    pl.kernel,
    out_type=jax.ShapeDtypeStruct((NC, NP, 16), jnp.float32),
    mesh=_mesh,
    scratch_types=[
        pltpu.VMEM((EW,), jnp.int32),
        pltpu.VMEM((K, 16), jnp.float32),
        pltpu.VMEM((RB, 16), jnp.float32),
        pltpu.VMEM_SHARED((NP, 16), jnp.float32),
        pltpu.SemaphoreType.DMA,
    ],
    compiler_params=pltpu.CompilerParams(use_tc_tiling_on_sc=False),
)
def _sc_degree(dst_hbm, ones_hbm, zero_hbm, out_hbm,
               dst_v, ones_v, stg_v, acc_sh, sem):
  c = lax.axis_index("c")
  s = lax.axis_index("s")
  wid = c * NS + s
  stripe = s * ROWS_W

  pltpu.sync_copy(zero_hbm, stg_v)
  for r in range(ROWS_W // RB):
    pltpu.sync_copy(stg_v, acc_sh.at[pl.ds(stripe + r * RB, RB)])
  pltpu.sync_copy(ones_hbm, ones_v)
  pltpu.sync_copy(dst_hbm.at[pl.ds(wid * EW, EW)], dst_v)
  plsc.subcore_barrier()

  def fire(i, carry):
    pltpu.async_copy(ones_v, acc_sh.at[dst_v.at[pl.ds(i * K, K)]], sem,
                     add=True)
    return carry

  def drain(i, carry):
    pltpu.make_async_copy(ones_hbm, ones_v, sem).wait()
    return carry

  lax.fori_loop(0, EW // K, fire, 0)
  lax.fori_loop(0, EW // K, drain, 0)
  plsc.subcore_barrier()

  for r in range(ROWS_W // RB):
    pltpu.sync_copy(acc_sh.at[pl.ds(stripe + r * RB, RB)], stg_v)
    pltpu.sync_copy(stg_v, out_hbm.at[c, pl.ds(stripe + r * RB, RB)])


_RB_TC = 256
_GRID = NP // _RB_TC


def _tc_first(dega, x_pad, W1):
  def body(deg_ref, x_ref, w_ref, g_ref, dis_ref):
    deg = 1.0 + deg_ref[0, :, 0:1] + deg_ref[1, :, 0:1]
    dis = lax.rsqrt(deg)
    dis_ref[...] = dis
    t = jnp.dot(x_ref[...], w_ref[...], preferred_element_type=jnp.float32)
    g_ref[...] = t * dis

  return pl.pallas_call(
      body,
      grid=(_GRID,),
      in_specs=[
          pl.BlockSpec((NC, _RB_TC, 16), lambda i: (0, i, 0)),
          pl.BlockSpec((_RB_TC, F), lambda i: (i, 0)),
          pl.BlockSpec((F, D1), lambda i: (0, 0)),
      ],
      out_specs=[
          pl.BlockSpec((_RB_TC, D1), lambda i: (i, 0)),
          pl.BlockSpec((_RB_TC, 1), lambda i: (i, 0)),
      ],
      out_shape=[
          jax.ShapeDtypeStruct((NP, D1), jnp.float32),
          jax.ShapeDtypeStruct((NP, 1), jnp.float32),
      ],
  )(dega, x_pad, W1)


def _tc_layer(S, g, dis, b, Wn, D, Dn):
  def body(s_ref, g_ref, dis_ref, b_ref, w_ref, go_ref):
    dis = dis_ref[...]
    h = jnp.maximum(dis * (s_ref[0] + s_ref[1] + g_ref[...]) + b_ref[...], 0.0)
    go_ref[...] = jnp.dot(
        h, w_ref[...], preferred_element_type=jnp.float32) * dis

  return pl.pallas_call(
      body,
      grid=(_GRID,),
      in_specs=[
          pl.BlockSpec((NC, _RB_TC, D), lambda i: (0, i, 0)),
          pl.BlockSpec((_RB_TC, D), lambda i: (i, 0)),
          pl.BlockSpec((_RB_TC, 1), lambda i: (i, 0)),
          pl.BlockSpec((1, D), lambda i: (0, 0)),
          pl.BlockSpec((D, Dn), lambda i: (0, 0)),
      ],
      out_specs=pl.BlockSpec((_RB_TC, Dn), lambda i: (i, 0)),
      out_shape=jax.ShapeDtypeStruct((NP, Dn), jnp.float32),
  )(S, g, dis, b, Wn)


def _tc_pool_classify(S, g, dis, b, batch3, Wc, bc):
  def body(s_ref, g_ref, dis_ref, b_ref, bt_ref, wc_ref, bc_ref, o_ref,
           pooled_ref, cnt_ref):
    i = pl.program_id(0)
    dis = dis_ref[...]
    h = jnp.maximum(dis * (s_ref[0] + s_ref[1] + g_ref[...]) + b_ref[...], 0.0)
    bt = bt_ref[0]
    gids = lax.broadcasted_iota(jnp.int32, (G, _RB_TC), 0)
    oh = (gids == bt).astype(jnp.float32)
    pooled_p = jnp.dot(oh, h, preferred_element_type=jnp.float32)
    cnt_p = jnp.sum(oh, axis=1, keepdims=True)

    @pl.when(i == 0)
    def _():
      pooled_ref[...] = jnp.zeros_like(pooled_ref)
      cnt_ref[...] = jnp.zeros_like(cnt_ref)

    pooled_ref[...] += pooled_p
    cnt_ref[...] += cnt_p

    @pl.when(i == _GRID - 1)
    def _():
      p = pooled_ref[...] / jnp.maximum(cnt_ref[...], 1.0)
      logits = jnp.dot(p, wc_ref[...], preferred_element_type=jnp.float32)
      logits = logits + bc_ref[...]
      m = jnp.max(logits, axis=1, keepdims=True)
      z = logits - m
      o_ref[...] = z - jnp.log(jnp.sum(jnp.exp(z), axis=1, keepdims=True))

  return pl.pallas_call(
      body,
      grid=(_GRID,),
      in_specs=[
          pl.BlockSpec((NC, _RB_TC, D3), lambda i: (0, i, 0)),
          pl.BlockSpec((_RB_TC, D3), lambda i: (i, 0)),
          pl.BlockSpec((_RB_TC, 1), lambda i: (i, 0)),
          pl.BlockSpec((1, D3), lambda i: (0, 0)),
          pl.BlockSpec((1, 1, _RB_TC), lambda i: (i, 0, 0)),
          pl.BlockSpec((D3, 2), lambda i: (0, 0)),
          pl.BlockSpec((1, 2), lambda i: (0, 0)),
      ],
      out_specs=pl.BlockSpec((G, 2), lambda i: (0, 0)),
      out_shape=jax.ShapeDtypeStruct((G, 2), jnp.float32),
      scratch_shapes=[
          pltpu.VMEM((G, D3), jnp.float32),
          pltpu.VMEM((G, 1), jnp.float32),
      ],
  )(S, g, dis, b, batch3, Wc, bc)


K1 = 40
_sc_l1 = _make_sc_layer(D1, K1, 5)
_sc_l2 = _make_sc_layer(D2, K, 10)
_sc_l3 = _make_sc_layer(D3, K, 10)


@jax.jit
def kernel(x, edge_index, batch, W1, b1, W2, b2, W3, b3, Wc, bc):
  src = edge_index[0]
  dst = edge_index[1]
  x_pad = jnp.zeros((NP, F), jnp.float32).at[:N].set(x)
  batch_pad = jnp.concatenate(
      [batch, jnp.full((NP - N,), G, jnp.int32)]).reshape(_GRID, 1, _RB_TC)

  ones80 = jnp.ones((K, 16), jnp.float32)
  zero16 = jnp.zeros((RB, 16), jnp.float32)
  z1 = jnp.zeros((K1, D1), jnp.float32)
  z2 = jnp.zeros((K, D2), jnp.float32)
  z3 = jnp.zeros((K, D3), jnp.float32)

  dega = _sc_degree(dst, ones80, zero16)
  g1, dis = _tc_first(dega, x_pad, W1)
  S1 = _sc_l1(src, dst, g1, z1)
  g2 = _tc_layer(S1, g1, dis, b1.reshape(1, D1), W2, D1, D2)
  S2 = _sc_l2(src, dst, g2, z2)
  g3 = _tc_layer(S2, g2, dis, b2.reshape(1, D2), W3, D2, D3)
  S3 = _sc_l3(src, dst, g3, z3)
  return _tc_pool_classify(S3, g3, dis, b3.reshape(1, D3), batch_pad,
                           Wc, bc.reshape(1, 2))
